# Initial kernel scaffold; baseline (speedup 1.0000x reference)
import jax
import jax.numpy as jnp
from jax.experimental import pallas as pl


def kernel(x, idx, W1, b1, W2, b2, W3, b3):
    raise NotImplementedError("write your pallas kernel here")


# speedup vs baseline: 7.5273x; 7.5273x over previous
import functools

import jax
import jax.numpy as jnp
from jax import lax
from jax.experimental import pallas as pl
from jax.experimental.pallas import tpu as pltpu
from jax.experimental.pallas import tpu_sc as plsc

B, C, N, K = 1024, 128, 24, 4
R = B * N
F = 64
NW = 32
BPW = B // NW
CB = 4
CR = CB * N
NCHUNK = BPW // CB
NGRP = CR // 16


def _mm_body(x_ref, w_ref, b_ref, o_ref):
    o_ref[...] = (
        jnp.dot(x_ref[...], w_ref[...], preferred_element_type=jnp.float32)
        + b_ref[...]
    )


def _matmul(x, w, b, blk=2048):
    rows, cin = x.shape
    cout = w.shape[1]
    return pl.pallas_call(
        _mm_body,
        grid=(rows // blk,),
        in_specs=[
            pl.BlockSpec((blk, cin), lambda i: (i, 0)),
            pl.BlockSpec((cin, cout), lambda i: (0, 0)),
            pl.BlockSpec((1, cout), lambda i: (0, 0)),
        ],
        out_specs=pl.BlockSpec((blk, cout), lambda i: (i, 0)),
        out_shape=jax.ShapeDtypeStruct((rows, cout), jnp.float32),
    )(x, w, b)


def _sc_gather_max(pq_flat, gt_flat):
    mesh = plsc.VectorSubcoreMesh(
        core_axis_name="c", subcore_axis_name="s", num_cores=2, num_subcores=16
    )

    @functools.partial(
        pl.kernel,
        out_type=jax.ShapeDtypeStruct((R * F,), jnp.float32),
        mesh=mesh,
        compiler_params=pltpu.CompilerParams(needs_layout_passes=False),
        scratch_types=[
            pltpu.VMEM((CR * 2 * F,), jnp.float32),
            pltpu.VMEM((K * CR,), jnp.int32),
            pltpu.VMEM((CR * F,), jnp.float32),
        ],
    )
    def run(pq_hbm, gt_hbm, out_hbm, pq_v, gt_v, h_v):
        wid = lax.axis_index("s") * 2 + lax.axis_index("c")
        lane = lax.iota(jnp.int32, 16)

        def chunk_body(c, carry):
            ch = wid * NCHUNK + c
            pltpu.sync_copy(pq_hbm.at[pl.ds(ch * CR * 2 * F, CR * 2 * F)], pq_v)
            pltpu.sync_copy(gt_hbm.at[pl.ds(ch * K * CR, K * CR)], gt_v)

            def grp_body(g, carry2):
                rl = g * 16 + lane
                rl2f = rl * (2 * F)
                rlf = rl * F
                lv = [
                    gt_v[pl.ds(kk * CR + g * 16, 16)] * (2 * F) for kk in range(K)
                ]
                for f in range(F):
                    m = plsc.load_gather(pq_v, [lv[0] + f])
                    for kk in range(1, K):
                        m = jnp.maximum(m, plsc.load_gather(pq_v, [lv[kk] + f]))
                    rv = plsc.load_gather(pq_v, [rl2f + (F + f)])
                    h = m + rv
                    h = jnp.where(h >= 0, h, 0.2 * h)
                    plsc.store_scatter(h_v, [rlf + f], h)
                return carry2

            lax.fori_loop(0, NGRP, grp_body, 0)
            pltpu.sync_copy(h_v, out_hbm.at[pl.ds(ch * CR * F, CR * F)])
            return carry

        lax.fori_loop(0, NCHUNK, chunk_body, 0)

    return run(pq_flat, gt_flat)


def _combine(W, b, cin):
    Wc = jnp.concatenate([W[:cin], W[cin:] - W[:cin]], axis=1)
    bc = jnp.concatenate([jnp.zeros((F,), jnp.float32), b]).reshape(1, 2 * F)
    return Wc, bc


def kernel(x, idx, W1, b1, W2, b2, W3, b3):
    xt = jnp.transpose(x, (0, 2, 1)).reshape(R, C)
    Wc1, bc1 = _combine(W1, b1, C)
    Wc2, bc2 = _combine(W2, b2, F)
    Wc3, bc3 = _combine(W3, b3, F)
    loc = (jnp.arange(B, dtype=jnp.int32) % CB)[:, None, None] * N + idx
    gt = jnp.transpose(loc.reshape(R // CR, CR, K), (0, 2, 1)).reshape(-1)

    pq1 = _matmul(xt, Wc1, bc1)
    h1 = _sc_gather_max(pq1.reshape(-1), gt).reshape(R, F)
    pq2 = _matmul(h1, Wc2, bc2)
    h2 = _sc_gather_max(pq2.reshape(-1), gt).reshape(R, F)
    pq3 = _matmul(h2, Wc3, bc3)
    h3 = _sc_gather_max(pq3.reshape(-1), gt).reshape(R, F)

    out = jnp.concatenate([h1, h2, h3], axis=1).reshape(B, N, 3 * F)
    return jnp.transpose(out, (0, 2, 1))


# speedup vs baseline: 14.7637x; 1.9614x over previous
import functools

import jax
import jax.numpy as jnp
from jax import lax
from jax.experimental import pallas as pl
from jax.experimental.pallas import tpu as pltpu
from jax.experimental.pallas import tpu_sc as plsc

B, C, N, K = 1024, 128, 24, 4
R = B * N
F = 64
NW = 32
BPW = B // NW
CB = 4
CR = CB * N
NCHUNK = BPW // CB
NGRP = CR // 16


def _mm_body(x_ref, w_ref, b_ref, o_ref):
    o_ref[...] = (
        jnp.dot(x_ref[...], w_ref[...], preferred_element_type=jnp.float32)
        + b_ref[...]
    )


def _matmul(x, w, b, blk=2048):
    rows, cin = x.shape
    cout = w.shape[1]
    return pl.pallas_call(
        _mm_body,
        grid=(rows // blk,),
        in_specs=[
            pl.BlockSpec((blk, cin), lambda i: (i, 0)),
            pl.BlockSpec((cin, cout), lambda i: (0, 0)),
            pl.BlockSpec((1, cout), lambda i: (0, 0)),
        ],
        out_specs=pl.BlockSpec((blk, cout), lambda i: (i, 0)),
        out_shape=jax.ShapeDtypeStruct((rows, cout), jnp.float32),
    )(x, w, b)


def _sc_gather_max(pq_flat, gtx_flat):
    mesh = plsc.VectorSubcoreMesh(
        core_axis_name="c", subcore_axis_name="s", num_cores=2, num_subcores=16
    )

    @functools.partial(
        pl.kernel,
        out_type=jax.ShapeDtypeStruct((R * F,), jnp.float32),
        mesh=mesh,
        compiler_params=pltpu.CompilerParams(needs_layout_passes=False),
        scratch_types=[
            pltpu.VMEM((CR * 2 * F,), jnp.float32),
            pltpu.VMEM((CR * K * 16,), jnp.int32),
            pltpu.VMEM((CR * F,), jnp.float32),
        ],
    )
    def run(pq_hbm, gtx_hbm, out_hbm, pq_v, gtx_v, h_v):
        wid = lax.axis_index("s") * 2 + lax.axis_index("c")

        def chunk_body(c, carry):
            ch = wid * NCHUNK + c
            pltpu.sync_copy(pq_hbm.at[pl.ds(ch * CR * 2 * F, CR * 2 * F)], pq_v)
            pltpu.sync_copy(
                gtx_hbm.at[pl.ds(ch * CR * K * 16, CR * K * 16)], gtx_v
            )

            def row_body(rr, carry2):
                gbase = rr * (K * 16)
                rbase = rr * (2 * F) + F
                hbase = rr * F
                bv = [gtx_v[pl.ds(gbase + kk * 16, 16)] for kk in range(K)]
                for f0 in range(0, F, 16):
                    m = plsc.load_gather(pq_v, [bv[0] + f0])
                    for kk in range(1, K):
                        m = jnp.maximum(
                            m, plsc.load_gather(pq_v, [bv[kk] + f0])
                        )
                    h = m + pq_v[pl.ds(rbase + f0, 16)]
                    h = jnp.where(h >= 0, h, 0.2 * h)
                    h_v[pl.ds(hbase + f0, 16)] = h
                return carry2

            lax.fori_loop(0, CR, row_body, 0)
            pltpu.sync_copy(h_v, out_hbm.at[pl.ds(ch * CR * F, CR * F)])
            return carry

        lax.fori_loop(0, NCHUNK, chunk_body, 0)

    return run(pq_flat, gtx_flat)


def _combine(W, b, cin):
    Wc = jnp.concatenate([W[:cin], W[cin:] - W[:cin]], axis=1)
    bc = jnp.concatenate([jnp.zeros((F,), jnp.float32), b]).reshape(1, 2 * F)
    return Wc, bc


def kernel(x, idx, W1, b1, W2, b2, W3, b3):
    xt = jnp.transpose(x, (0, 2, 1)).reshape(R, C)
    Wc1, bc1 = _combine(W1, b1, C)
    Wc2, bc2 = _combine(W2, b2, F)
    Wc3, bc3 = _combine(W3, b3, F)
    loc = (jnp.arange(B, dtype=jnp.int32) % CB)[:, None, None] * N + idx
    gtx = (
        loc.reshape(R, K)[:, :, None] * (2 * F)
        + jnp.arange(16, dtype=jnp.int32)
    ).reshape(-1)

    pq1 = _matmul(xt, Wc1, bc1)
    h1 = _sc_gather_max(pq1.reshape(-1), gtx).reshape(R, F)
    pq2 = _matmul(h1, Wc2, bc2)
    h2 = _sc_gather_max(pq2.reshape(-1), gtx).reshape(R, F)
    pq3 = _matmul(h2, Wc3, bc3)
    h3 = _sc_gather_max(pq3.reshape(-1), gtx).reshape(R, F)

    out = jnp.concatenate([h1, h2, h3], axis=1).reshape(B, N, 3 * F)
    return jnp.transpose(out, (0, 2, 1))
